# baseline (speedup 1.0000x reference)
import jax
import jax.numpy as jnp
from jax import lax
from jax.experimental import pallas as pl
from jax.experimental.pallas import tpu as pltpu

_B, _N, _C = 8, 2048, 131
_NPT = 512


def _fps_kernel(x2d_ref, xt_ref, out_ref, dists_ref, asq_ref):
    iota8 = lax.broadcasted_iota(jnp.int32, (_B, 1), 0)

    for b in range(_B):
        xb = xt_ref[b]
        asq_ref[b:b + 1, :] = jnp.sum(xb * xb, axis=0, keepdims=True)
    dists_ref[...] = jnp.full((_B, _N), 1e10, jnp.float32)

    def _scal(vec, b):
        return jnp.max(jnp.where(iota8 == b, vec, -1))

    def body(i, carry):
        fs, nfv = carry
        iota = lax.broadcasted_iota(jnp.int32, (_B, _N), 1)
        iota1 = lax.broadcasted_iota(jnp.int32, (1, _N), 1)
        sqf = jnp.max(jnp.where(iota == nfv, asq_ref[...], -jnp.inf),
                      axis=1, keepdims=True)
        nf_bs = []
        for b in range(_B):
            out_ref[i, b] = fs[b]
            row = x2d_ref[pl.ds(b * _N + fs[b], 1), :]
            corr = lax.dot_general(
                row, xt_ref[b],
                dimension_numbers=(((1,), (0,)), ((), ())),
                preferred_element_type=jnp.float32,
            )
            d = (sqf[b:b + 1, :] + asq_ref[b:b + 1, :]) - 2.0 * corr
            nd = jnp.minimum(dists_ref[b:b + 1, :], d)
            dists_ref[b:b + 1, :] = nd
            m = jnp.max(nd)
            nf_bs.append(jnp.min(jnp.where(nd == m, iota1, _N), axis=1,
                                 keepdims=True).astype(jnp.int32))
        nnfv = jnp.concatenate(nf_bs, axis=0)
        nfs = tuple(_scal(nnfv, b) for b in range(_B))
        return nfs, nnfv

    fs0 = tuple(jnp.int32(0) for _ in range(_B))
    nfv0 = jnp.zeros((_B, 1), jnp.int32)
    lax.fori_loop(0, _NPT, body, (fs0, nfv0))


def kernel(points_xyz, features):
    feats_t = jnp.transpose(features, (0, 2, 1))
    xcat = jnp.concatenate([points_xyz, feats_t], axis=2)
    x2d = xcat.reshape(_B * _N, _C)
    xt = jnp.concatenate(
        [jnp.transpose(points_xyz, (0, 2, 1)), features], axis=1)
    out = pl.pallas_call(
        _fps_kernel,
        out_shape=jax.ShapeDtypeStruct((_NPT, _B), jnp.int32),
        in_specs=[
            pl.BlockSpec(memory_space=pltpu.VMEM),
            pl.BlockSpec(memory_space=pltpu.VMEM),
        ],
        out_specs=pl.BlockSpec(memory_space=pltpu.SMEM),
        scratch_shapes=[
            pltpu.VMEM((_B, _N), jnp.float32),
            pltpu.VMEM((_B, _N), jnp.float32),
        ],
    )(x2d, xt)
    return jnp.transpose(out, (1, 0))


# speedup vs baseline: 1.1627x; 1.1627x over previous
import jax
import jax.numpy as jnp
from jax import lax
from jax.experimental import pallas as pl
from jax.experimental.pallas import tpu as pltpu

_B, _N, _C = 8, 2048, 131
_NPT = 512


def _fps_kernel(x2d_ref, xt_ref, out_ref, corr_ref, asq_ref):
    iota8 = lax.broadcasted_iota(jnp.int32, (_B, 1), 0)

    for b in range(_B):
        xb = xt_ref[b]
        asq_ref[b:b + 1, :] = jnp.sum(xb * xb, axis=0, keepdims=True)

    def _scal(vec, b):
        return jnp.max(jnp.where(iota8 == b, vec, -1))

    def body(i, carry):
        fs, sqf, dists = carry
        iota = lax.broadcasted_iota(jnp.int32, (_B, _N), 1)
        for b in range(_B):
            out_ref[i, b] = fs[b]
            row = x2d_ref[pl.ds(b * _N + fs[b], 1), :]
            corr_ref[b:b + 1, :] = lax.dot_general(
                row, xt_ref[b],
                dimension_numbers=(((1,), (0,)), ((), ())),
                preferred_element_type=jnp.float32,
            )
        corr = corr_ref[...]
        a_sq = asq_ref[...]
        d = (sqf + a_sq) - 2.0 * corr
        nd = jnp.minimum(dists, d)
        m = jnp.max(nd, axis=1, keepdims=True)
        nf = jnp.min(jnp.where(nd == m, iota, _N), axis=1,
                     keepdims=True).astype(jnp.int32)
        nsqf = jnp.max(jnp.where(iota == nf, a_sq, -jnp.inf), axis=1,
                       keepdims=True)
        nfs = tuple(_scal(nf, b) for b in range(_B))
        return nfs, nsqf, nd

    a_sq0 = asq_ref[...]
    iota_full = lax.broadcasted_iota(jnp.int32, (_B, _N), 1)
    fs0 = tuple(jnp.int32(0) for _ in range(_B))
    sqf0 = jnp.max(jnp.where(iota_full == 0, a_sq0, -jnp.inf), axis=1,
                   keepdims=True)
    dists0 = jnp.full((_B, _N), 1e10, jnp.float32)
    lax.fori_loop(0, _NPT, body, (fs0, sqf0, dists0))


def kernel(points_xyz, features):
    feats_t = jnp.transpose(features, (0, 2, 1))
    xcat = jnp.concatenate([points_xyz, feats_t], axis=2)
    x2d = xcat.reshape(_B * _N, _C)
    xt = jnp.concatenate(
        [jnp.transpose(points_xyz, (0, 2, 1)), features], axis=1)
    out = pl.pallas_call(
        _fps_kernel,
        out_shape=jax.ShapeDtypeStruct((_NPT, _B), jnp.int32),
        in_specs=[
            pl.BlockSpec(memory_space=pltpu.VMEM),
            pl.BlockSpec(memory_space=pltpu.VMEM),
        ],
        out_specs=pl.BlockSpec(memory_space=pltpu.SMEM),
        scratch_shapes=[
            pltpu.VMEM((_B, _N), jnp.float32),
            pltpu.VMEM((_B, _N), jnp.float32),
        ],
    )(x2d, xt)
    return jnp.transpose(out, (1, 0))


# speedup vs baseline: 1.1889x; 1.0225x over previous
import jax
import jax.numpy as jnp
from jax import lax
from jax.experimental import pallas as pl
from jax.experimental.pallas import tpu as pltpu

_B, _N, _C = 8, 2048, 131
_NPT = 512


def _fps_kernel(x2d_ref, xt_ref, out_ref, corr_ref, asq_ref):
    iota8 = lax.broadcasted_iota(jnp.int32, (_B, 1), 0)

    for b in range(_B):
        xb = xt_ref[b]
        asq_ref[b:b + 1, :] = jnp.sum(xb * xb, axis=0, keepdims=True)

    def _scal(vec, b):
        return jnp.max(jnp.where(iota8 == b, vec, -1))

    def _step(i, carry):
        fs, sqf, dists = carry
        iota = lax.broadcasted_iota(jnp.int32, (_B, _N), 1)
        for b in range(_B):
            out_ref[i, b] = fs[b]
            row = x2d_ref[pl.ds(b * _N + fs[b], 1), :]
            corr_ref[b:b + 1, :] = lax.dot_general(
                row, xt_ref[b],
                dimension_numbers=(((1,), (0,)), ((), ())),
                preferred_element_type=jnp.float32,
            )
        corr = corr_ref[...]
        a_sq = asq_ref[...]
        d = (sqf + a_sq) - 2.0 * corr
        nd = jnp.minimum(dists, d)
        m = jnp.max(nd, axis=1, keepdims=True)
        nf = jnp.min(jnp.where(nd == m, iota, _N), axis=1,
                     keepdims=True).astype(jnp.int32)
        nsqf = jnp.max(jnp.where(iota == nf, a_sq, -jnp.inf), axis=1,
                       keepdims=True)
        nfs = tuple(_scal(nf, b) for b in range(_B))
        return nfs, nsqf, nd

    def body(i2, carry):
        return _step(2 * i2 + 1, _step(2 * i2, carry))

    a_sq0 = asq_ref[...]
    iota_full = lax.broadcasted_iota(jnp.int32, (_B, _N), 1)
    fs0 = tuple(jnp.int32(0) for _ in range(_B))
    sqf0 = jnp.max(jnp.where(iota_full == 0, a_sq0, -jnp.inf), axis=1,
                   keepdims=True)
    dists0 = jnp.full((_B, _N), 1e10, jnp.float32)
    lax.fori_loop(0, _NPT // 2, body, (fs0, sqf0, dists0))


def kernel(points_xyz, features):
    feats_t = jnp.transpose(features, (0, 2, 1))
    xcat = jnp.concatenate([points_xyz, feats_t], axis=2)
    x2d = xcat.reshape(_B * _N, _C)
    xt = jnp.concatenate(
        [jnp.transpose(points_xyz, (0, 2, 1)), features], axis=1)
    out = pl.pallas_call(
        _fps_kernel,
        out_shape=jax.ShapeDtypeStruct((_NPT, _B), jnp.int32),
        in_specs=[
            pl.BlockSpec(memory_space=pltpu.VMEM),
            pl.BlockSpec(memory_space=pltpu.VMEM),
        ],
        out_specs=pl.BlockSpec(memory_space=pltpu.SMEM),
        scratch_shapes=[
            pltpu.VMEM((_B, _N), jnp.float32),
            pltpu.VMEM((_B, _N), jnp.float32),
        ],
    )(x2d, xt)
    return jnp.transpose(out, (1, 0))


# speedup vs baseline: 1.2087x; 1.0167x over previous
import jax
import jax.numpy as jnp
from jax import lax
from jax.experimental import pallas as pl
from jax.experimental.pallas import tpu as pltpu

_B, _N, _C = 8, 2048, 131
_NPT = 512


def _fps_kernel(x2d_ref, xt_ref, out_ref, corr_ref, asq_ref):
    iota8 = lax.broadcasted_iota(jnp.int32, (_B, 1), 0)

    for b in range(_B):
        xb = xt_ref[b]
        asq_ref[b:b + 1, :] = jnp.sum(xb * xb, axis=0, keepdims=True)

    def _scal(vec, b):
        return jnp.max(jnp.where(iota8 == b, vec, -1))

    def _step(i, carry):
        fs, sqf, dists = carry
        iota = lax.broadcasted_iota(jnp.int32, (_B, _N), 1)
        for b in range(_B):
            out_ref[i, b] = fs[b]
            row = x2d_ref[pl.ds(b * _N + fs[b], 1), :]
            corr_ref[b:b + 1, :] = lax.dot_general(
                row, xt_ref[b],
                dimension_numbers=(((1,), (0,)), ((), ())),
                preferred_element_type=jnp.float32,
            )
        corr = corr_ref[...]
        a_sq = asq_ref[...]
        d = (sqf + a_sq) - 2.0 * corr
        nd = jnp.minimum(dists, d)
        m = jnp.max(nd, axis=1, keepdims=True)
        nf = jnp.min(jnp.where(nd == m, iota, _N), axis=1,
                     keepdims=True).astype(jnp.int32)
        nsqf = jnp.max(jnp.where(iota == nf, a_sq, -jnp.inf), axis=1,
                       keepdims=True)
        nfs = tuple(_scal(nf, b) for b in range(_B))
        return nfs, nsqf, nd

    def body(i4, carry):
        for k in range(4):
            carry = _step(4 * i4 + k, carry)
        return carry

    a_sq0 = asq_ref[...]
    iota_full = lax.broadcasted_iota(jnp.int32, (_B, _N), 1)
    fs0 = tuple(jnp.int32(0) for _ in range(_B))
    sqf0 = jnp.max(jnp.where(iota_full == 0, a_sq0, -jnp.inf), axis=1,
                   keepdims=True)
    dists0 = jnp.full((_B, _N), 1e10, jnp.float32)
    lax.fori_loop(0, _NPT // 4, body, (fs0, sqf0, dists0))


def kernel(points_xyz, features):
    feats_t = jnp.transpose(features, (0, 2, 1))
    xcat = jnp.concatenate([points_xyz, feats_t], axis=2)
    x2d = xcat.reshape(_B * _N, _C)
    xt = jnp.concatenate(
        [jnp.transpose(points_xyz, (0, 2, 1)), features], axis=1)
    out = pl.pallas_call(
        _fps_kernel,
        out_shape=jax.ShapeDtypeStruct((_NPT, _B), jnp.int32),
        in_specs=[
            pl.BlockSpec(memory_space=pltpu.VMEM),
            pl.BlockSpec(memory_space=pltpu.VMEM),
        ],
        out_specs=pl.BlockSpec(memory_space=pltpu.SMEM),
        scratch_shapes=[
            pltpu.VMEM((_B, _N), jnp.float32),
            pltpu.VMEM((_B, _N), jnp.float32),
        ],
    )(x2d, xt)
    return jnp.transpose(out, (1, 0))


# speedup vs baseline: 1.2213x; 1.0104x over previous
import jax
import jax.numpy as jnp
from jax import lax
from jax.experimental import pallas as pl
from jax.experimental.pallas import tpu as pltpu

_B, _N, _C = 8, 2048, 131
_NPT = 512


def _fps_kernel(x2d_ref, xt_ref, out_ref, corr_ref, asq_ref):
    iota8 = lax.broadcasted_iota(jnp.int32, (_B, 1), 0)

    for b in range(_B):
        xb = xt_ref[b]
        asq_ref[b:b + 1, :] = jnp.sum(xb * xb, axis=0, keepdims=True)

    def _scal(vec, b):
        return jnp.max(jnp.where(iota8 == b, vec, -1))

    def _step(i, carry):
        fs, nfv, dists = carry
        iota = lax.broadcasted_iota(jnp.int32, (_B, _N), 1)
        a_sq = asq_ref[...]
        sqf = jnp.max(jnp.where(iota == nfv, a_sq, -jnp.inf), axis=1,
                      keepdims=True)
        for b in range(_B):
            out_ref[i, b] = fs[b]
            row = x2d_ref[pl.ds(b * _N + fs[b], 1), :]
            corr_ref[b:b + 1, :] = lax.dot_general(
                row, xt_ref[b],
                dimension_numbers=(((1,), (0,)), ((), ())),
                preferred_element_type=jnp.float32,
            )
        corr = corr_ref[...]
        d = (sqf + a_sq) - 2.0 * corr
        nd = jnp.minimum(dists, d)
        m = jnp.max(nd, axis=1, keepdims=True)
        nf = jnp.min(jnp.where(nd == m, iota, _N), axis=1,
                     keepdims=True).astype(jnp.int32)
        nfs = tuple(_scal(nf, b) for b in range(_B))
        return nfs, nf, nd

    def body(i4, carry):
        for k in range(4):
            carry = _step(4 * i4 + k, carry)
        return carry

    fs0 = tuple(jnp.int32(0) for _ in range(_B))
    nfv0 = jnp.zeros((_B, 1), jnp.int32)
    dists0 = jnp.full((_B, _N), 1e10, jnp.float32)
    lax.fori_loop(0, _NPT // 4, body, (fs0, nfv0, dists0))


def kernel(points_xyz, features):
    feats_t = jnp.transpose(features, (0, 2, 1))
    xcat = jnp.concatenate([points_xyz, feats_t], axis=2)
    x2d = xcat.reshape(_B * _N, _C)
    xt = jnp.concatenate(
        [jnp.transpose(points_xyz, (0, 2, 1)), features], axis=1)
    out = pl.pallas_call(
        _fps_kernel,
        out_shape=jax.ShapeDtypeStruct((_NPT, _B), jnp.int32),
        in_specs=[
            pl.BlockSpec(memory_space=pltpu.VMEM),
            pl.BlockSpec(memory_space=pltpu.VMEM),
        ],
        out_specs=pl.BlockSpec(memory_space=pltpu.SMEM),
        scratch_shapes=[
            pltpu.VMEM((_B, _N), jnp.float32),
            pltpu.VMEM((_B, _N), jnp.float32),
        ],
    )(x2d, xt)
    return jnp.transpose(out, (1, 0))


# speedup vs baseline: 1.3205x; 1.0812x over previous
import jax
import jax.numpy as jnp
from jax import lax
from jax.experimental import pallas as pl
from jax.experimental.pallas import tpu as pltpu

_B, _N, _C = 8, 2048, 131
_NPT = 512


def _fps_kernel(x2d_ref, xt_ref, out_ref, corr_ref, asq_ref):
    iota8 = lax.broadcasted_iota(jnp.int32, (_B, 1), 0)

    for b in range(_B):
        xb = xt_ref[b]
        asq_ref[b:b + 1, :] = jnp.sum(xb * xb, axis=0, keepdims=True)

    def _scal(vec, b):
        return jnp.max(jnp.where(iota8 == b, vec, -1))

    def _step(i, carry):
        fs, nfv, dists = carry
        iota = lax.broadcasted_iota(jnp.int32, (_B, _N), 1)
        a_sq = asq_ref[...]
        sqf = jnp.max(jnp.where(iota == nfv, a_sq, -jnp.inf), axis=1,
                      keepdims=True)
        for b in range(_B):
            out_ref[i, b] = fs[b]
            row = x2d_ref[pl.ds(b * _N + fs[b], 1), :]
            corr_ref[b:b + 1, :] = lax.dot_general(
                row, xt_ref[b],
                dimension_numbers=(((1,), (0,)), ((), ())),
                preferred_element_type=jnp.float32,
            )
        corr = corr_ref[...]
        d = (sqf + a_sq) - 2.0 * corr
        nd = jnp.minimum(dists, d)
        nf = jnp.argmax(nd, axis=1, keepdims=True).astype(jnp.int32)
        nfs = tuple(_scal(nf, b) for b in range(_B))
        return nfs, nf, nd

    def body(i4, carry):
        for k in range(4):
            carry = _step(4 * i4 + k, carry)
        return carry

    fs0 = tuple(jnp.int32(0) for _ in range(_B))
    nfv0 = jnp.zeros((_B, 1), jnp.int32)
    dists0 = jnp.full((_B, _N), 1e10, jnp.float32)
    lax.fori_loop(0, _NPT // 4, body, (fs0, nfv0, dists0))


def kernel(points_xyz, features):
    feats_t = jnp.transpose(features, (0, 2, 1))
    xcat = jnp.concatenate([points_xyz, feats_t], axis=2)
    x2d = xcat.reshape(_B * _N, _C)
    xt = jnp.concatenate(
        [jnp.transpose(points_xyz, (0, 2, 1)), features], axis=1)
    out = pl.pallas_call(
        _fps_kernel,
        out_shape=jax.ShapeDtypeStruct((_NPT, _B), jnp.int32),
        in_specs=[
            pl.BlockSpec(memory_space=pltpu.VMEM),
            pl.BlockSpec(memory_space=pltpu.VMEM),
        ],
        out_specs=pl.BlockSpec(memory_space=pltpu.SMEM),
        scratch_shapes=[
            pltpu.VMEM((_B, _N), jnp.float32),
            pltpu.VMEM((_B, _N), jnp.float32),
        ],
    )(x2d, xt)
    return jnp.transpose(out, (1, 0))


# speedup vs baseline: 1.6291x; 1.2337x over previous
import jax
import jax.numpy as jnp
from jax import lax
from jax.experimental import pallas as pl
from jax.experimental.pallas import tpu as pltpu

_B, _N, _C = 8, 2048, 131
_G = 4
_NPT = 512


def _fps_kernel(x2d_ref, xt_ref, out_ref, corr_a_ref, corr_b_ref, asq_a_ref,
                asq_b_ref):
    corr_refs = (corr_a_ref, corr_b_ref)
    asq_refs = (asq_a_ref, asq_b_ref)
    iotag = lax.broadcasted_iota(jnp.int32, (_G, 1), 0)

    for b in range(_B):
        xb = xt_ref[b]
        asq_refs[b // _G][b % _G:b % _G + 1, :] = jnp.sum(
            xb * xb, axis=0, keepdims=True)
    for b in range(_B):
        row = x2d_ref[b * _N:b * _N + 1, :]
        corr_refs[b // _G][b % _G:b % _G + 1, :] = lax.dot_general(
            row, xt_ref[b],
            dimension_numbers=(((1,), (0,)), ((), ())),
            preferred_element_type=jnp.float32,
        )

    def _scal(vec, b):
        return jnp.max(jnp.where(iotag == b, vec, -1))

    def _phase(i, g, fs, nfv, dists):
        cref = corr_refs[g]
        iota = lax.broadcasted_iota(jnp.int32, (_G, _N), 1)
        a_sq = asq_refs[g][...]
        sqf = jnp.max(jnp.where(iota == nfv, a_sq, -jnp.inf), axis=1,
                      keepdims=True)
        corr = cref[...]
        d = (sqf + a_sq) - 2.0 * corr
        nd = jnp.minimum(dists, d)
        nf = jnp.argmax(nd, axis=1, keepdims=True).astype(jnp.int32)
        nfs = tuple(_scal(nf, b) for b in range(_G))
        for b in range(_G):
            bb = g * _G + b
            out_ref[i, bb] = fs[b]
            row = x2d_ref[pl.ds(bb * _N + nfs[b], 1), :]
            cref[b:b + 1, :] = lax.dot_general(
                row, xt_ref[bb],
                dimension_numbers=(((1,), (0,)), ((), ())),
                preferred_element_type=jnp.float32,
            )
        return nfs, nf, nd

    def body(i4, carry):
        fs_a, fs_b, nfv_a, nfv_b, d_a, d_b = carry
        for k in range(4):
            i = 4 * i4 + k
            fs_a, nfv_a, d_a = _phase(i, 0, fs_a, nfv_a, d_a)
            fs_b, nfv_b, d_b = _phase(i, 1, fs_b, nfv_b, d_b)
        return fs_a, fs_b, nfv_a, nfv_b, d_a, d_b

    fs0 = tuple(jnp.int32(0) for _ in range(_G))
    nfv0 = jnp.zeros((_G, 1), jnp.int32)
    dists0 = jnp.full((_G, _N), 1e10, jnp.float32)
    lax.fori_loop(0, _NPT // 4, body,
                  (fs0, fs0, nfv0, nfv0, dists0, dists0))


def kernel(points_xyz, features):
    feats_t = jnp.transpose(features, (0, 2, 1))
    xcat = jnp.concatenate([points_xyz, feats_t], axis=2)
    x2d = xcat.reshape(_B * _N, _C)
    xt = jnp.concatenate(
        [jnp.transpose(points_xyz, (0, 2, 1)), features], axis=1)
    out = pl.pallas_call(
        _fps_kernel,
        out_shape=jax.ShapeDtypeStruct((_NPT, _B), jnp.int32),
        in_specs=[
            pl.BlockSpec(memory_space=pltpu.VMEM),
            pl.BlockSpec(memory_space=pltpu.VMEM),
        ],
        out_specs=pl.BlockSpec(memory_space=pltpu.SMEM),
        scratch_shapes=[
            pltpu.VMEM((_G, _N), jnp.float32),
            pltpu.VMEM((_G, _N), jnp.float32),
            pltpu.VMEM((_G, _N), jnp.float32),
            pltpu.VMEM((_G, _N), jnp.float32),
        ],
    )(x2d, xt)
    return jnp.transpose(out, (1, 0))
